# baseline (speedup 1.0000x reference)
import functools

import jax
import jax.numpy as jnp
from jax.experimental import pallas as pl
from jax.experimental.pallas import tpu as pltpu


def _prep_body(x_ref, wrel_ref, hrel_ref):
    hrel_ref[...] = jnp.dot(x_ref[...], wrel_ref[...],
                            preferred_element_type=jnp.float32)


def _fused_body(adj_ref, x_ref, hrel_ref, wself_ref, wq_ref, wk_ref,
                watt_ref, bias_ref, o_ref):
    self_ft = jnp.dot(x_ref[...], wself_ref[...],
                      preferred_element_type=jnp.float32)
    nb = jnp.dot(adj_ref[...], hrel_ref[...],
                 preferred_element_type=jnp.float32)

    att_q = jnp.dot(self_ft, wq_ref[...], preferred_element_type=jnp.float32)
    att_k0 = jnp.dot(self_ft, wk_ref[...], preferred_element_type=jnp.float32)
    att_k1 = jnp.dot(nb, wk_ref[...], preferred_element_type=jnp.float32)

    ai0 = jnp.concatenate([att_k0, att_q], axis=1)
    ai1 = jnp.concatenate([att_k1, att_q], axis=1)
    watt = watt_ref[...]
    v0 = jnp.dot(ai0, watt, preferred_element_type=jnp.float32)
    v1 = jnp.dot(ai1, watt, preferred_element_type=jnp.float32)
    e0 = jnp.where(v0 > 0, v0, jnp.exp(jnp.minimum(v0, 0.0)) - 1.0)
    e1 = jnp.where(v1 > 0, v1, jnp.exp(jnp.minimum(v1, 0.0)) - 1.0)

    m = jnp.maximum(e0, e1)
    z0 = jnp.exp(e0 - m)
    z1 = jnp.exp(e1 - m)
    denom = z0 + z1
    a0 = z0 / denom
    a1 = z1 / denom

    o_ref[...] = self_ft * a0 + nb * a1 + bias_ref[...]


@jax.jit
def kernel(x_dict, adj_dict, W_rel, w_self, bias, w_query, w_keys, w_att):
    N, DIN = x_dict.shape
    DOUT = W_rel.shape[1]
    T2 = w_att.shape[0]

    BA = 1000
    hrel = pl.pallas_call(
        _prep_body,
        grid=(N // BA,),
        in_specs=[
            pl.BlockSpec((BA, DIN), lambda i: (i, 0)),
            pl.BlockSpec((DIN, DOUT), lambda i: (0, 0)),
        ],
        out_specs=pl.BlockSpec((BA, DOUT), lambda i: (i, 0)),
        out_shape=jax.ShapeDtypeStruct((N, DOUT), jnp.float32),
        compiler_params=pltpu.CompilerParams(
            dimension_semantics=("arbitrary",)),
    )(x_dict, W_rel)

    BN = 200
    out = pl.pallas_call(
        _fused_body,
        grid=(N // BN,),
        in_specs=[
            pl.BlockSpec((BN, N), lambda i: (i, 0)),
            pl.BlockSpec((BN, DIN), lambda i: (i, 0)),
            pl.BlockSpec((N, DOUT), lambda i: (0, 0)),
            pl.BlockSpec((DIN, DOUT), lambda i: (0, 0)),
            pl.BlockSpec(w_query.shape, lambda i: (0, 0)),
            pl.BlockSpec(w_keys.shape, lambda i: (0, 0)),
            pl.BlockSpec((T2, 1), lambda i: (0, 0)),
            pl.BlockSpec((1, DOUT), lambda i: (0, 0)),
        ],
        out_specs=pl.BlockSpec((BN, DOUT), lambda i: (i, 0)),
        out_shape=jax.ShapeDtypeStruct((N, DOUT), jnp.float32),
        compiler_params=pltpu.CompilerParams(
            dimension_semantics=("parallel",),
            vmem_limit_bytes=100 * 1024 * 1024),
    )(adj_dict, x_dict, hrel, w_self, w_query, w_keys, w_att, bias)
    return out


# speedup vs baseline: 1.1015x; 1.1015x over previous
import functools

import jax
import jax.numpy as jnp
from jax.experimental import pallas as pl
from jax.experimental.pallas import tpu as pltpu


def _prep_body(x_ref, wrel_ref, hrel_ref):
    hrel_ref[...] = jnp.dot(x_ref[...], wrel_ref[...],
                            preferred_element_type=jnp.float32)


def _fused_body(adj_ref, x_ref, hrel_ref, wself_ref, wq_ref, wk_ref,
                watt_ref, bias_ref, o_ref):
    self_ft = jnp.dot(x_ref[...], wself_ref[...],
                      preferred_element_type=jnp.float32)
    nb = jnp.dot(adj_ref[...], hrel_ref[...],
                 preferred_element_type=jnp.float32)

    att_q = jnp.dot(self_ft, wq_ref[...], preferred_element_type=jnp.float32)
    att_k0 = jnp.dot(self_ft, wk_ref[...], preferred_element_type=jnp.float32)
    att_k1 = jnp.dot(nb, wk_ref[...], preferred_element_type=jnp.float32)

    ai0 = jnp.concatenate([att_k0, att_q], axis=1)
    ai1 = jnp.concatenate([att_k1, att_q], axis=1)
    watt = watt_ref[...]
    v0 = jnp.dot(ai0, watt, preferred_element_type=jnp.float32)
    v1 = jnp.dot(ai1, watt, preferred_element_type=jnp.float32)
    e0 = jnp.where(v0 > 0, v0, jnp.exp(jnp.minimum(v0, 0.0)) - 1.0)
    e1 = jnp.where(v1 > 0, v1, jnp.exp(jnp.minimum(v1, 0.0)) - 1.0)

    m = jnp.maximum(e0, e1)
    z0 = jnp.exp(e0 - m)
    z1 = jnp.exp(e1 - m)
    denom = z0 + z1
    a0 = z0 / denom
    a1 = z1 / denom

    o_ref[...] = self_ft * a0 + nb * a1 + bias_ref[...]


@jax.jit
def kernel(x_dict, adj_dict, W_rel, w_self, bias, w_query, w_keys, w_att):
    N, DIN = x_dict.shape
    DOUT = W_rel.shape[1]
    T2 = w_att.shape[0]

    BA = 1000
    hrel = pl.pallas_call(
        _prep_body,
        grid=(N // BA,),
        in_specs=[
            pl.BlockSpec((BA, DIN), lambda i: (i, 0)),
            pl.BlockSpec((DIN, DOUT), lambda i: (0, 0)),
        ],
        out_specs=pl.BlockSpec((BA, DOUT), lambda i: (i, 0)),
        out_shape=jax.ShapeDtypeStruct((N, DOUT), jnp.float32),
        compiler_params=pltpu.CompilerParams(
            dimension_semantics=("arbitrary",)),
    )(x_dict, W_rel)

    BN = 400
    out = pl.pallas_call(
        _fused_body,
        grid=(N // BN,),
        in_specs=[
            pl.BlockSpec((BN, N), lambda i: (i, 0)),
            pl.BlockSpec((BN, DIN), lambda i: (i, 0)),
            pl.BlockSpec((N, DOUT), lambda i: (0, 0)),
            pl.BlockSpec((DIN, DOUT), lambda i: (0, 0)),
            pl.BlockSpec(w_query.shape, lambda i: (0, 0)),
            pl.BlockSpec(w_keys.shape, lambda i: (0, 0)),
            pl.BlockSpec((T2, 1), lambda i: (0, 0)),
            pl.BlockSpec((1, DOUT), lambda i: (0, 0)),
        ],
        out_specs=pl.BlockSpec((BN, DOUT), lambda i: (i, 0)),
        out_shape=jax.ShapeDtypeStruct((N, DOUT), jnp.float32),
        compiler_params=pltpu.CompilerParams(
            dimension_semantics=("parallel",),
            vmem_limit_bytes=64 * 1024 * 1024),
    )(adj_dict, x_dict, hrel, w_self, w_query, w_keys, w_att, bias)
    return out


# speedup vs baseline: 1.1445x; 1.0390x over previous
import functools

import jax
import jax.numpy as jnp
from jax.experimental import pallas as pl
from jax.experimental.pallas import tpu as pltpu


def _prep_body(x_ref, wrel_ref, hrel_ref):
    hrel_ref[...] = jnp.dot(x_ref[...], wrel_ref[...],
                            preferred_element_type=jnp.float32
                            ).astype(jnp.bfloat16)


def _fused_body(adj_ref, x_ref, hrel_ref, wself_ref, wq_ref, wk_ref,
                watt_ref, bias_ref, o_ref):
    self_ft = jnp.dot(x_ref[...], wself_ref[...],
                      preferred_element_type=jnp.float32)
    nb = jnp.dot(adj_ref[...].astype(jnp.bfloat16), hrel_ref[...],
                 preferred_element_type=jnp.float32)

    att_q = jnp.dot(self_ft, wq_ref[...], preferred_element_type=jnp.float32)
    att_k0 = jnp.dot(self_ft, wk_ref[...], preferred_element_type=jnp.float32)
    att_k1 = jnp.dot(nb, wk_ref[...], preferred_element_type=jnp.float32)

    ai0 = jnp.concatenate([att_k0, att_q], axis=1)
    ai1 = jnp.concatenate([att_k1, att_q], axis=1)
    watt = watt_ref[...]
    v0 = jnp.dot(ai0, watt, preferred_element_type=jnp.float32)
    v1 = jnp.dot(ai1, watt, preferred_element_type=jnp.float32)
    e0 = jnp.where(v0 > 0, v0, jnp.exp(jnp.minimum(v0, 0.0)) - 1.0)
    e1 = jnp.where(v1 > 0, v1, jnp.exp(jnp.minimum(v1, 0.0)) - 1.0)

    m = jnp.maximum(e0, e1)
    z0 = jnp.exp(e0 - m)
    z1 = jnp.exp(e1 - m)
    denom = z0 + z1
    a0 = z0 / denom
    a1 = z1 / denom

    o_ref[...] = self_ft * a0 + nb * a1 + bias_ref[...]


@jax.jit
def kernel(x_dict, adj_dict, W_rel, w_self, bias, w_query, w_keys, w_att):
    N, DIN = x_dict.shape
    DOUT = W_rel.shape[1]
    T2 = w_att.shape[0]

    BA = 1000
    hrel = pl.pallas_call(
        _prep_body,
        grid=(N // BA,),
        in_specs=[
            pl.BlockSpec((BA, DIN), lambda i: (i, 0)),
            pl.BlockSpec((DIN, DOUT), lambda i: (0, 0)),
        ],
        out_specs=pl.BlockSpec((BA, DOUT), lambda i: (i, 0)),
        out_shape=jax.ShapeDtypeStruct((N, DOUT), jnp.bfloat16),
        compiler_params=pltpu.CompilerParams(
            dimension_semantics=("arbitrary",)),
    )(x_dict, W_rel)

    BN = 400
    out = pl.pallas_call(
        _fused_body,
        grid=(N // BN,),
        in_specs=[
            pl.BlockSpec((BN, N), lambda i: (i, 0)),
            pl.BlockSpec((BN, DIN), lambda i: (i, 0)),
            pl.BlockSpec((N, DOUT), lambda i: (0, 0)),
            pl.BlockSpec((DIN, DOUT), lambda i: (0, 0)),
            pl.BlockSpec(w_query.shape, lambda i: (0, 0)),
            pl.BlockSpec(w_keys.shape, lambda i: (0, 0)),
            pl.BlockSpec((T2, 1), lambda i: (0, 0)),
            pl.BlockSpec((1, DOUT), lambda i: (0, 0)),
        ],
        out_specs=pl.BlockSpec((BN, DOUT), lambda i: (i, 0)),
        out_shape=jax.ShapeDtypeStruct((N, DOUT), jnp.float32),
        compiler_params=pltpu.CompilerParams(
            dimension_semantics=("parallel",),
            vmem_limit_bytes=64 * 1024 * 1024),
    )(adj_dict, x_dict, hrel, w_self, w_query, w_keys, w_att, bias)
    return out


# speedup vs baseline: 1.1462x; 1.0015x over previous
import functools

import jax
import jax.numpy as jnp
from jax.experimental import pallas as pl
from jax.experimental.pallas import tpu as pltpu


def _fused_body(adj_ref, x_ref, wrel_ref, wself_ref, wq_ref, wk_ref,
                watt_ref, bias_ref, o_ref, hrel_ref):
    @pl.when(pl.program_id(0) == 0)
    def _():
        n = x_ref.shape[0]
        chunk = n // 10
        for k in range(10):
            rows = pl.ds(k * chunk, chunk)
            hrel_ref[rows, :] = jnp.dot(x_ref[rows, :], wrel_ref[...],
                                        preferred_element_type=jnp.float32
                                        ).astype(jnp.bfloat16)

    i = pl.program_id(0)
    bn = o_ref.shape[0]
    x_blk = x_ref[pl.ds(i * bn, bn), :]
    self_ft = jnp.dot(x_blk, wself_ref[...],
                      preferred_element_type=jnp.float32)
    nb = jnp.dot(adj_ref[...].astype(jnp.bfloat16), hrel_ref[...],
                 preferred_element_type=jnp.float32)

    att_q = jnp.dot(self_ft, wq_ref[...], preferred_element_type=jnp.float32)
    att_k0 = jnp.dot(self_ft, wk_ref[...], preferred_element_type=jnp.float32)
    att_k1 = jnp.dot(nb, wk_ref[...], preferred_element_type=jnp.float32)

    ai0 = jnp.concatenate([att_k0, att_q], axis=1)
    ai1 = jnp.concatenate([att_k1, att_q], axis=1)
    watt = watt_ref[...]
    v0 = jnp.dot(ai0, watt, preferred_element_type=jnp.float32)
    v1 = jnp.dot(ai1, watt, preferred_element_type=jnp.float32)
    e0 = jnp.where(v0 > 0, v0, jnp.exp(jnp.minimum(v0, 0.0)) - 1.0)
    e1 = jnp.where(v1 > 0, v1, jnp.exp(jnp.minimum(v1, 0.0)) - 1.0)

    m = jnp.maximum(e0, e1)
    z0 = jnp.exp(e0 - m)
    z1 = jnp.exp(e1 - m)
    denom = z0 + z1
    a0 = z0 / denom
    a1 = z1 / denom

    o_ref[...] = self_ft * a0 + nb * a1 + bias_ref[...]


@jax.jit
def kernel(x_dict, adj_dict, W_rel, w_self, bias, w_query, w_keys, w_att):
    N, DIN = x_dict.shape
    DOUT = W_rel.shape[1]
    T2 = w_att.shape[0]

    x_bf = x_dict.astype(jnp.bfloat16)
    wrel_bf = W_rel.astype(jnp.bfloat16)

    BN = 400
    out = pl.pallas_call(
        _fused_body,
        grid=(N // BN,),
        in_specs=[
            pl.BlockSpec((BN, N), lambda i: (i, 0)),
            pl.BlockSpec((N, DIN), lambda i: (0, 0)),
            pl.BlockSpec((DIN, DOUT), lambda i: (0, 0)),
            pl.BlockSpec((DIN, DOUT), lambda i: (0, 0)),
            pl.BlockSpec(w_query.shape, lambda i: (0, 0)),
            pl.BlockSpec(w_keys.shape, lambda i: (0, 0)),
            pl.BlockSpec((T2, 1), lambda i: (0, 0)),
            pl.BlockSpec((1, DOUT), lambda i: (0, 0)),
        ],
        out_specs=pl.BlockSpec((BN, DOUT), lambda i: (i, 0)),
        out_shape=jax.ShapeDtypeStruct((N, DOUT), jnp.float32),
        scratch_shapes=[pltpu.VMEM((N, DOUT), jnp.bfloat16)],
        compiler_params=pltpu.CompilerParams(
            dimension_semantics=("arbitrary",),
            vmem_limit_bytes=64 * 1024 * 1024),
    )(adj_dict, x_bf, wrel_bf, w_self, w_query, w_keys, w_att, bias)
    return out


# speedup vs baseline: 1.1824x; 1.0316x over previous
import functools

import jax
import jax.numpy as jnp
from jax.experimental import pallas as pl
from jax.experimental.pallas import tpu as pltpu

_PREP = 10


def _fused_body(adj_ref, xc_ref, xb_ref, wrel_ref, wself_ref, wq_ref,
                wk_ref, watt_ref, bias_ref, o_ref, hrel_ref):
    i = pl.program_id(0)
    ch = xc_ref.shape[0]

    @pl.when(i < _PREP)
    def _():
        hrel_ref[pl.ds(i * ch, ch), :] = jnp.dot(
            xc_ref[...], wrel_ref[...],
            preferred_element_type=jnp.float32)

    @pl.when(i >= _PREP)
    def _():
        self_ft = jnp.dot(xb_ref[...], wself_ref[...],
                          preferred_element_type=jnp.float32)
        nb = jnp.dot(adj_ref[...], hrel_ref[...],
                     preferred_element_type=jnp.float32)

        att_q = jnp.dot(self_ft, wq_ref[...],
                        preferred_element_type=jnp.float32)
        att_k0 = jnp.dot(self_ft, wk_ref[...],
                         preferred_element_type=jnp.float32)
        att_k1 = jnp.dot(nb, wk_ref[...],
                         preferred_element_type=jnp.float32)

        ai0 = jnp.concatenate([att_k0, att_q], axis=1)
        ai1 = jnp.concatenate([att_k1, att_q], axis=1)
        watt = watt_ref[...]
        v0 = jnp.dot(ai0, watt, preferred_element_type=jnp.float32)
        v1 = jnp.dot(ai1, watt, preferred_element_type=jnp.float32)
        e0 = jnp.where(v0 > 0, v0, jnp.exp(jnp.minimum(v0, 0.0)) - 1.0)
        e1 = jnp.where(v1 > 0, v1, jnp.exp(jnp.minimum(v1, 0.0)) - 1.0)

        m = jnp.maximum(e0, e1)
        z0 = jnp.exp(e0 - m)
        z1 = jnp.exp(e1 - m)
        denom = z0 + z1
        a0 = z0 / denom
        a1 = z1 / denom

        o_ref[...] = self_ft * a0 + nb * a1 + bias_ref[...]


@jax.jit
def kernel(x_dict, adj_dict, W_rel, w_self, bias, w_query, w_keys, w_att):
    N, DIN = x_dict.shape
    DOUT = W_rel.shape[1]
    T2 = w_att.shape[0]

    BN = 400
    CH = N // _PREP

    agg = lambda i: (jnp.maximum(i - _PREP, 0), 0)
    out = pl.pallas_call(
        _fused_body,
        grid=(_PREP + N // BN,),
        in_specs=[
            pl.BlockSpec((BN, N), agg),
            pl.BlockSpec((CH, DIN),
                         lambda i: (jnp.minimum(i, _PREP - 1), 0)),
            pl.BlockSpec((BN, DIN), agg),
            pl.BlockSpec((DIN, DOUT), lambda i: (0, 0)),
            pl.BlockSpec((DIN, DOUT), lambda i: (0, 0)),
            pl.BlockSpec(w_query.shape, lambda i: (0, 0)),
            pl.BlockSpec(w_keys.shape, lambda i: (0, 0)),
            pl.BlockSpec((T2, 1), lambda i: (0, 0)),
            pl.BlockSpec((1, DOUT), lambda i: (0, 0)),
        ],
        out_specs=pl.BlockSpec((BN, DOUT), agg),
        out_shape=jax.ShapeDtypeStruct((N, DOUT), jnp.float32),
        scratch_shapes=[pltpu.VMEM((N, DOUT), jnp.float32)],
        compiler_params=pltpu.CompilerParams(
            dimension_semantics=("arbitrary",),
            vmem_limit_bytes=64 * 1024 * 1024),
    )(adj_dict, x_dict, x_dict, W_rel, w_self, w_query, w_keys, w_att, bias)
    return out
